# Initial kernel scaffold; baseline (speedup 1.0000x reference)
import functools

import jax
import jax.numpy as jnp
from jax import lax
from jax.experimental import pallas as pl
from jax.experimental.pallas import tpu as pltpu
from jax.experimental.pallas import tpu_sc as plsc

VOX = 0.4
XMIN, YMIN, ZMIN = 0.0, -51.2, -4.0
NX, NY, NZ = 256, 256, 20
MAX_VOX = 16000
MAX_PTS = 3
BIG = NX * NY * NZ
BIGP = BIG + 32
B, N = 4, 200000
NPAD = 200704
CHUNK = NPAD // 32
NRNG = 256
RSZ = BIG // NRNG
SLOTP = 16384
SW = 512
INF = jnp.int32(2**31 - 1)

MESH = plsc.VectorSubcoreMesh(core_axis_name="c", subcore_axis_name="s")
CP = pltpu.CompilerParams(needs_layout_passes=False)


def _wid():
    return lax.axis_index("s") * 2 + lax.axis_index("c")


def _i16():
    return lax.iota(jnp.int32, 16)


def _extract(buf, i):
    v = plsc.load_gather(buf, [jnp.full((16,), i, jnp.int32)])
    return jnp.max(v)


@functools.partial(
    pl.kernel,
    out_type=(
        jax.ShapeDtypeStruct((B, 32, 49, 128), jnp.int32),
        jax.ShapeDtypeStruct((2, B, BIG), jnp.int32),
    ),
    mesh=MESH,
    compiler_params=CP,
    scratch_types=[
        pltpu.VMEM_SHARED((BIGP,), jnp.int32),
        pltpu.VMEM((CHUNK,), jnp.float32),
        pltpu.VMEM((CHUNK,), jnp.float32),
        pltpu.VMEM((CHUNK,), jnp.float32),
        pltpu.VMEM((49, 128), jnp.int32),
        pltpu.VMEM((128,), jnp.int32),
        pltpu.VMEM((16384,), jnp.int32),
    ],
)
def _k1(xc, yc, zc, lin_o, cnt_o, spc, xb, yb, zb, lin2, ones, zbuf):
    cid = lax.axis_index("c")
    sid = lax.axis_index("s")
    wid = _wid()
    i16 = _i16()

    def init(i, _):
        zbuf[pl.ds(i * 16, 16)] = jnp.zeros((16,), jnp.int32)
        return 0

    lax.fori_loop(0, 1024, init, 0, unroll=False)
    for j in range(8):
        ones[pl.ds(j * 16, 16)] = jnp.ones((16,), jnp.int32)

    def coord(vb, off, lo):
        v = (vb[pl.ds(off, 16)] - lo) / jnp.float32(VOX)
        ci = v.astype(jnp.int32)
        ci = ci - (ci.astype(jnp.float32) > v).astype(jnp.int32)
        return ci

    def batch(b, _):
        plsc.subcore_barrier()
        def z5(j, _):
            pltpu.sync_copy(zbuf, spc.at[pl.ds(sid * 81920 + j * 16384, 16384)])
            return 0

        lax.fori_loop(0, 5, z5, 0, unroll=False)

        @pl.when(sid == 0)
        def _():
            pltpu.sync_copy(zbuf.at[pl.ds(0, 32)], spc.at[pl.ds(BIG, 32)])

        plsc.subcore_barrier()
        base = wid * CHUNK
        pltpu.sync_copy(xc.at[b, pl.ds(base, CHUNK)], xb)
        pltpu.sync_copy(yc.at[b, pl.ds(base, CHUNK)], yb)
        pltpu.sync_copy(zc.at[b, pl.ds(base, CHUNK)], zb)

        def row(j, _):
            for v in range(8):
                o = j * 128 + v * 16
                cx = coord(xb, o, jnp.float32(XMIN))
                cy = coord(yb, o, jnp.float32(YMIN))
                cz = coord(zb, o, jnp.float32(ZMIN))
                inr = ((cx >= 0) & (cx < NX) & (cy >= 0) & (cy < NY)
                       & (cz >= 0) & (cz < NZ))
                lv = (cz * NY + cy) * NX + cx
                lv = jnp.where(inr, lv, BIG + i16)
                lin2[j, pl.ds(v * 16, 16)] = lv
            pltpu.sync_copy(ones, spc.at[lin2.at[j]], add=True)
            return 0

        lax.fori_loop(0, 49, row, 0, unroll=False)
        pltpu.sync_copy(lin2, lin_o.at[b, wid])
        plsc.subcore_barrier()
        pltpu.sync_copy(spc.at[pl.ds(sid * 81920, 81920)],
                        cnt_o.at[cid, b, pl.ds(sid * 81920, 81920)])
        return 0

    lax.fori_loop(0, B, batch, 0, unroll=False)


@functools.partial(
    pl.kernel,
    out_type=jax.ShapeDtypeStruct((B, NRNG, 16), jnp.int32),
    mesh=MESH,
    compiler_params=CP,
    scratch_types=[
        pltpu.VMEM((RSZ,), jnp.int32),
        pltpu.VMEM((RSZ,), jnp.int32),
        pltpu.VMEM((16,), jnp.int32),
    ],
)
def _k2a(cnt_i, tot_o, ba, bb, t16):
    wid = _wid()

    def batch(b, _):
        def rng(k, _):
            r = wid + k * 32
            pltpu.sync_copy(cnt_i.at[0, b, pl.ds(r * RSZ, RSZ)], ba)
            pltpu.sync_copy(cnt_i.at[1, b, pl.ds(r * RSZ, RSZ)], bb)

            def scan(j, acc):
                ab = ba[pl.ds(j * 16, 16)] + bb[pl.ds(j * 16, 16)]
                return acc + (ab > 0).astype(jnp.int32)

            acc = lax.fori_loop(0, RSZ // 16, scan, jnp.zeros((16,), jnp.int32),
                                unroll=False)
            t16[...] = jnp.full((16,), jnp.sum(acc), jnp.int32)
            pltpu.sync_copy(t16, tot_o.at[b, r])
            return 0

        lax.fori_loop(0, 8, rng, 0, unroll=False)
        return 0

    lax.fori_loop(0, B, batch, 0, unroll=False)


@functools.partial(
    pl.kernel,
    out_type=(
        jax.ShapeDtypeStruct((B * BIGP,), jnp.int32),
        jax.ShapeDtypeStruct((B, SLOTP, 8), jnp.int32),
        jax.ShapeDtypeStruct((B, SLOTP, 8), jnp.int32),
    ),
    mesh=MESH,
    compiler_params=CP,
    scratch_types=[
        pltpu.VMEM((4096,), jnp.int32),
        pltpu.VMEM((256,), jnp.int32),
        pltpu.VMEM((RSZ,), jnp.int32),
        pltpu.VMEM((RSZ,), jnp.int32),
        pltpu.VMEM((RSZ,), jnp.int32),
        pltpu.VMEM((4096,), jnp.int32),
        pltpu.VMEM((4096,), jnp.int32),
        pltpu.VMEM((4096,), jnp.int32),
        pltpu.VMEM((4096,), jnp.int32),
    ],
)
def _k2b(cnt_i, tot_i, sd_o, cp_o, vc_o, tbuf, pbuf, ba, bb, sdb, infb,
         cslab, vslab, zbuf):
    wid = _wid()
    i16 = _i16()

    def init(i, _):
        infb[pl.ds(i * 16, 16)] = jnp.full((16,), INF, jnp.int32)
        zbuf[pl.ds(i * 16, 16)] = jnp.zeros((16,), jnp.int32)
        return 0

    lax.fori_loop(0, 256, init, 0, unroll=False)

    def batch(b, _):
        def ld16(j, _):
            pltpu.sync_copy(tot_i.at[b, pl.ds(j * 16, 16)].reshape(256),
                            tbuf.at[pl.ds(j * 256, 256)])
            return 0

        lax.fori_loop(0, 16, ld16, 0, unroll=False)

        def pfx(j, carry):
            tv = plsc.load_gather(tbuf, [(j * 16 + i16) * 16])
            cs = plsc.cumsum(tv)
            pbuf[pl.ds(j * 16, 16)] = carry + cs - tv
            return carry + jnp.max(cs)

        total = lax.fori_loop(0, 16, pfx, jnp.int32(0), unroll=False)

        def rng(k, _):
            r = wid + k * 32
            base = _extract(pbuf, r)

            @pl.when(base >= MAX_VOX)
            def _():
                def fi(j, _):
                    pltpu.sync_copy(
                        infb,
                        sd_o.at[pl.ds(b * BIGP + r * RSZ + j * 4096, 4096)])
                    return 0

                lax.fori_loop(0, RSZ // 4096, fi, 0, unroll=False)

            @pl.when(base < MAX_VOX)
            def _():
                pltpu.sync_copy(cnt_i.at[0, b, pl.ds(r * RSZ, RSZ)], ba)
                pltpu.sync_copy(cnt_i.at[1, b, pl.ds(r * RSZ, RSZ)], bb)

                def scan(j, carry):
                    ab = ba[pl.ds(j * 16, 16)] + bb[pl.ds(j * 16, 16)]
                    occ = (ab > 0).astype(jnp.int32)
                    cs = plsc.cumsum(occ)
                    slot = base + carry + cs - occ
                    sdv = jnp.where((occ > 0) & (slot < MAX_VOX), slot, INF)
                    sdb[pl.ds(j * 16, 16)] = sdv
                    return carry + jnp.max(cs)

                lax.fori_loop(0, RSZ // 16, scan, jnp.int32(0), unroll=False)
                pltpu.sync_copy(sdb, sd_o.at[pl.ds(b * BIGP + r * RSZ, RSZ)])
            return 0

        lax.fori_loop(0, 8, rng, 0, unroll=False)

        @pl.when(wid == 0)
        def _():
            pltpu.sync_copy(infb.at[pl.ds(0, 32)],
                            sd_o.at[pl.ds(b * BIGP + BIG, 32)])

        pltpu.sync_copy(zbuf, cslab)
        pltpu.sync_copy(zbuf, vslab)
        slo = wid * SW
        shi = slo + SW

        @pl.when(slo < total)
        def _():
            def rng2(r, _):
                base = _extract(pbuf, r)
                tr = _extract(tbuf, r * 16)

                @pl.when((base < shi) & (base + tr > slo))
                def _():
                    pltpu.sync_copy(cnt_i.at[0, b, pl.ds(r * RSZ, RSZ)], ba)
                    pltpu.sync_copy(cnt_i.at[1, b, pl.ds(r * RSZ, RSZ)], bb)

                    def scan(j, carry):
                        ab = ba[pl.ds(j * 16, 16)] + bb[pl.ds(j * 16, 16)]
                        occ = (ab > 0).astype(jnp.int32)
                        cs = plsc.cumsum(occ)
                        slot = base + carry + cs - occ
                        mine = ((occ > 0) & (slot >= slo) & (slot < shi)
                                & (slot < MAX_VOX))
                        loc8 = (slot - slo) * 8
                        lv = r * RSZ + j * 16 + i16
                        plsc.store_scatter(cslab, [loc8],
                                           jnp.minimum(ab, MAX_PTS), mask=mine)
                        plsc.store_scatter(vslab, [loc8], lv >> 16, mask=mine)
                        plsc.store_scatter(vslab, [loc8 + 1], (lv >> 8) & 255,
                                           mask=mine)
                        plsc.store_scatter(vslab, [loc8 + 2], lv & 255,
                                           mask=mine)
                        return carry + jnp.max(cs)

                    lax.fori_loop(0, RSZ // 16, scan, jnp.int32(0),
                                  unroll=False)
                return 0

            lax.fori_loop(0, NRNG, rng2, 0, unroll=False)

        cpf = cp_o.reshape(B * SLOTP * 8)
        vcf = vc_o.reshape(B * SLOTP * 8)
        pltpu.sync_copy(cslab, cpf.at[pl.ds((b * SLOTP + slo) * 8, 4096)])
        pltpu.sync_copy(vslab, vcf.at[pl.ds((b * SLOTP + slo) * 8, 4096)])
        return 0

    lax.fori_loop(0, B, batch, 0, unroll=False)


@functools.partial(
    pl.kernel,
    out_type=(
        jax.ShapeDtypeStruct((B, 32, CHUNK), jnp.int32),
        jax.ShapeDtypeStruct((B, 32, 16), jnp.int32),
    ),
    mesh=MESH,
    compiler_params=CP,
    scratch_types=[
        pltpu.VMEM((49, 128), jnp.int32),
        pltpu.VMEM((49, 128), jnp.int32),
        pltpu.VMEM((CHUNK,), jnp.int32),
        pltpu.VMEM((16,), jnp.int32),
        pltpu.SemaphoreType.DMA,
    ],
)
def _k3a(lin_i, sd_i, flt_o, fc_o, l2, sd2, fbuf, t16, sem):
    wid = _wid()
    i16 = _i16()

    def batch(b, _):
        pltpu.sync_copy(lin_i.at[b, wid], l2)

        def grow(j, _):
            pltpu.async_copy(sd_i.at[pl.ds(b * BIGP, BIGP)].at[l2.at[j]],
                             sd2.at[j], sem).wait()
            return 0

        lax.fori_loop(0, 49, grow, 0, unroll=False)

        def row(j, off):
            for v in range(8):
                o = j * 128 + v * 16
                sdv = sd2[j, pl.ds(v * 16, 16)]
                lv = l2[j, pl.ds(v * 16, 16)]
                keep = (sdv < MAX_VOX) & (lv < BIG)
                pidx = wid * CHUNK + o + i16
                packed = lax.shift_left(pidx, 14) | jnp.where(keep, sdv, 0)
                plsc.store_compressed(fbuf.at[pl.ds(off, 16)], packed,
                                      mask=keep)
                off = off + jnp.sum(keep.astype(jnp.int32))
            return off

        off = lax.fori_loop(0, 49, row, jnp.int32(0), unroll=False)
        pltpu.sync_copy(fbuf, flt_o.at[b, wid])
        t16[...] = jnp.full((16,), off, jnp.int32)
        pltpu.sync_copy(t16, fc_o.at[b, wid])
        return 0

    lax.fori_loop(0, B, batch, 0, unroll=False)


@functools.partial(
    pl.kernel,
    out_type=(
        jax.ShapeDtypeStruct((B, SLOTP, 16), jnp.float32),
        jax.ShapeDtypeStruct((B, SLOTP, 8), jnp.float32),
    ),
    mesh=MESH,
    compiler_params=CP,
    scratch_types=[
        pltpu.VMEM((8192,), jnp.float32),
        pltpu.VMEM((512,), jnp.int32),
        pltpu.VMEM((512,), jnp.int32),
        pltpu.VMEM((512,), jnp.int32),
        pltpu.VMEM((1568,), jnp.int32),
        pltpu.VMEM((1568,), jnp.int32),
        pltpu.VMEM((16,), jnp.float32),
        pltpu.VMEM((4096,), jnp.int32),
        pltpu.VMEM((4096,), jnp.float32),
        pltpu.VMEM((4096,), jnp.float32),
        pltpu.SemaphoreType.DMA,
    ],
)
def _k3c(flt_i, fc_i, cp_i, feat_i, vf_o, enc_o, slab, cloc, sbuf, fc, kidx,
         kfeat, fb, cbuf, ebuf, zf, sem):
    wid = _wid()
    i16 = _i16()
    slo = wid * SW

    def init(i, _):
        zf[pl.ds(i * 16, 16)] = jnp.zeros((16,), jnp.float32)
        return 0

    lax.fori_loop(0, 256, init, 0, unroll=False)

    def batch(b, _):
        pltpu.sync_copy(zf, slab.at[pl.ds(0, 4096)])
        pltpu.sync_copy(zf, slab.at[pl.ds(4096, 4096)])

        def zc(i, _):
            cloc[pl.ds(i * 16, 16)] = jnp.zeros((16,), jnp.int32)
            return 0

        lax.fori_loop(0, 32, zc, 0, unroll=False)

        def ldfc(j, _):
            pltpu.sync_copy(fc_i.at[b, pl.ds(j * 16, 16)].reshape(256),
                            fc.at[pl.ds(j * 256, 256)])
            return 0

        lax.fori_loop(0, 2, ldfc, 0, unroll=False)

        def chunk(c, koff):
            nc = _extract(fc, c * 16)

            def tile(t, koff):
                pltpu.sync_copy(flt_i.at[b, c, pl.ds(t * 512, 512)], sbuf)

                def vr(j, koff):
                    e = sbuf[pl.ds(j * 16, 16)]
                    pos = t * 512 + j * 16 + i16
                    sdv = e & 0x3FFF
                    pidx = lax.shift_right_logical(e, 14)
                    mine = (pos < nc) & (sdv >= slo) & (sdv < slo + SW)
                    sl = jnp.where(mine, sdv - slo, 511)
                    dup, last = plsc.scan_count(sl, mask=mine)
                    c0 = plsc.load_gather(cloc, [sl], mask=mine)
                    rank = c0 + dup - 1
                    keep = mine & (rank < MAX_PTS)
                    plsc.store_scatter(cloc, [sl],
                                       jnp.minimum(c0 + dup, MAX_PTS),
                                       mask=mine & last)
                    nk = jnp.sum(keep.astype(jnp.int32))

                    @pl.when(nk > 0)
                    def _():
                        plsc.store_compressed(
                            kidx.at[pl.ds(koff, 16)],
                            sl * 16 + rank * 5, mask=keep)
                        plsc.store_compressed(
                            kfeat.at[pl.ds(koff, 16)],
                            (b * NPAD + pidx) * 5, mask=keep)
                    return koff + nk

                return lax.fori_loop(0, 32, vr, koff, unroll=False)

            nt = (nc + 511) // 512
            return lax.fori_loop(0, nt, tile, koff, unroll=False)

        koff = lax.fori_loop(0, 32, chunk, jnp.int32(0), unroll=False)

        def grp(g, _):
            ki = kidx[pl.ds(g * 16, 16)]
            kf = kfeat[pl.ds(g * 16, 16)]
            valid = g * 16 + i16 < koff
            kfs = jnp.where(valid, kf, 0)
            pltpu.async_copy(feat_i.at[kfs], fb, sem).wait()
            plsc.store_scatter(slab, [jnp.where(valid, ki, 8191)], fb[...],
                               mask=valid)
            for c5 in range(1, 5):
                pltpu.async_copy(feat_i.at[kfs + c5], fb, sem).wait()
                plsc.store_scatter(slab, [jnp.where(valid, ki + c5, 8191)],
                                   fb[...], mask=valid)
            return 0

        lax.fori_loop(0, (koff + 15) // 16, grp, 0, unroll=False)
        vff = vf_o.reshape(B * SLOTP * 16)
        pltpu.sync_copy(slab.at[pl.ds(0, 4096)],
                        vff.at[pl.ds((b * SLOTP + slo) * 16, 4096)])
        pltpu.sync_copy(slab.at[pl.ds(4096, 4096)],
                        vff.at[pl.ds((b * SLOTP + slo) * 16 + 4096, 4096)])

        cpf = cp_i.reshape(B * SLOTP * 8)
        pltpu.sync_copy(cpf.at[pl.ds((b * SLOTP + slo) * 8, 4096)], cbuf)

        def enc(j, _):
            sl16 = j * 16 + i16
            cnt = plsc.load_gather(cbuf, [sl16 * 8])
            den = jnp.maximum(cnt, 1).astype(jnp.float32)
            for c5 in range(5):
                s = (plsc.load_gather(slab, [sl16 * 16 + c5])
                     + plsc.load_gather(slab, [sl16 * 16 + 5 + c5])
                     + plsc.load_gather(slab, [sl16 * 16 + 10 + c5]))
                plsc.store_scatter(ebuf, [sl16 * 8 + c5], s / den)
            return 0

        lax.fori_loop(0, 32, enc, 0, unroll=False)
        encf = enc_o.reshape(B * SLOTP * 8)
        pltpu.sync_copy(ebuf, encf.at[pl.ds((b * SLOTP + slo) * 8, 4096)])
        return 0

    lax.fori_loop(0, B, batch, 0, unroll=False)


def kernel(sparse_cube, sparse_cube_dop, batch_size):
    cat = jnp.concatenate([sparse_cube, sparse_cube_dop[:, :, 3:4]], axis=-1)
    pad = NPAD - N
    catp = jnp.pad(cat, ((0, 0), (0, pad), (0, 0)), constant_values=-10.0)
    xc = catp[:, :, 0]
    yc = catp[:, :, 1]
    zc = catp[:, :, 2]
    feat_flat = catp.reshape(B * NPAD * 5)

    lin, cnt2 = _k1(xc, yc, zc)
    tot = _k2a(cnt2)
    sd, cp, vc = _k2b(cnt2, tot)
    flt, fcn = _k3a(lin, sd)
    vf_pad, enc_pad = _k3c(flt, fcn, cp, feat_flat)

    encoded = enc_pad[:, :MAX_VOX, :5].reshape(B * MAX_VOX, 5)
    voxel_features = vf_pad[:, :MAX_VOX, :15].reshape(B * MAX_VOX, MAX_PTS, 5)
    counts = cp[:, :MAX_VOX, 0].reshape(B * MAX_VOX)
    bcol = jnp.minimum(jnp.arange(B, dtype=jnp.int32),
                       jnp.asarray(batch_size - 1, jnp.int32))
    bcol = jnp.repeat(bcol, MAX_VOX)[:, None]
    voxel_coords = jnp.concatenate(
        [bcol, vc[:, :MAX_VOX, :3].reshape(B * MAX_VOX, 3)], axis=1)
    pts_idx = jnp.repeat(jnp.arange(B), N).astype(cat.dtype)
    points = jnp.concatenate([pts_idx[:, None], cat.reshape(B * N, 5)], axis=-1)
    return encoded, voxel_features, voxel_coords, counts, points


# speedup vs baseline: 30.8989x; 30.8989x over previous
import functools

import jax
import jax.numpy as jnp
from jax import lax
from jax.experimental import pallas as pl
from jax.experimental.pallas import tpu as pltpu
from jax.experimental.pallas import tpu_sc as plsc

VOX = 0.4
XMIN, YMIN, ZMIN = 0.0, -51.2, -4.0
NX, NY, NZ = 256, 256, 20
MAX_VOX = 16000
MAX_PTS = 3
BIG = NX * NY * NZ
BIGP = BIG + 32
B, N = 4, 200000
NPAD = 200704
CHUNK = NPAD // 32
NRNG = 256
RSZ = BIG // NRNG
SLOTP = 16384
SW = 512
INF = 2**31 - 1

MESH = plsc.VectorSubcoreMesh(core_axis_name="c", subcore_axis_name="s")
CP = pltpu.CompilerParams(needs_layout_passes=False)


def _wid():
    return lax.axis_index("s") * 2 + lax.axis_index("c")


def _i16():
    return lax.iota(jnp.int32, 16)


def _extract(buf, i):
    v = plsc.load_gather(buf, [jnp.full((16,), i, jnp.int32)])
    return jnp.max(v)


@functools.partial(
    pl.kernel,
    out_type=(
        jax.ShapeDtypeStruct((B, 32, 49, 128), jnp.int32),
        jax.ShapeDtypeStruct((2, B, BIG), jnp.int32),
    ),
    mesh=MESH,
    compiler_params=CP,
    scratch_types=[
        pltpu.VMEM_SHARED((BIGP,), jnp.int32),
        pltpu.VMEM((CHUNK,), jnp.float32),
        pltpu.VMEM((CHUNK,), jnp.float32),
        pltpu.VMEM((CHUNK,), jnp.float32),
        pltpu.VMEM((49, 128), jnp.int32),
        pltpu.VMEM((128,), jnp.int32),
        pltpu.VMEM((16384,), jnp.int32),
    ],
)
def _k1(xc, yc, zc, lin_o, cnt_o, spc, xb, yb, zb, lin2, ones, zbuf):
    cid = lax.axis_index("c")
    sid = lax.axis_index("s")
    wid = _wid()
    i16 = _i16()

    def init(i, _):
        zbuf[pl.ds(i * 16, 16)] = jnp.zeros((16,), jnp.int32)
        return 0

    lax.fori_loop(0, 1024, init, 0, unroll=False)
    for j in range(8):
        ones[pl.ds(j * 16, 16)] = jnp.ones((16,), jnp.int32)

    def coord(vb, off, lo):
        v = (vb[pl.ds(off, 16)] - lo) / jnp.float32(VOX)
        ci = v.astype(jnp.int32)
        ci = ci - (ci.astype(jnp.float32) > v).astype(jnp.int32)
        return ci

    def batch(b, _):
        plsc.subcore_barrier()
        def z5(j, _):
            pltpu.sync_copy(zbuf, spc.at[pl.ds(sid * 81920 + j * 16384, 16384)])
            return 0

        lax.fori_loop(0, 5, z5, 0, unroll=False)

        @pl.when(sid == 0)
        def _():
            pltpu.sync_copy(zbuf.at[pl.ds(0, 32)], spc.at[pl.ds(BIG, 32)])

        plsc.subcore_barrier()
        base = wid * CHUNK
        pltpu.sync_copy(xc.at[b, pl.ds(base, CHUNK)], xb)
        pltpu.sync_copy(yc.at[b, pl.ds(base, CHUNK)], yb)
        pltpu.sync_copy(zc.at[b, pl.ds(base, CHUNK)], zb)

        def row(j, _):
            for v in range(8):
                o = j * 128 + v * 16
                cx = coord(xb, o, jnp.float32(XMIN))
                cy = coord(yb, o, jnp.float32(YMIN))
                cz = coord(zb, o, jnp.float32(ZMIN))
                inr = ((cx >= 0) & (cx < NX) & (cy >= 0) & (cy < NY)
                       & (cz >= 0) & (cz < NZ))
                lv = (cz * NY + cy) * NX + cx
                lv = jnp.where(inr, lv, BIG + i16)
                lin2[j, pl.ds(v * 16, 16)] = lv
            pltpu.sync_copy(ones, spc.at[lin2.at[j]], add=True)
            return 0

        lax.fori_loop(0, 49, row, 0, unroll=False)
        pltpu.sync_copy(lin2, lin_o.at[b, wid])
        plsc.subcore_barrier()
        pltpu.sync_copy(spc.at[pl.ds(sid * 81920, 81920)],
                        cnt_o.at[cid, b, pl.ds(sid * 81920, 81920)])
        return 0

    lax.fori_loop(0, B, batch, 0, unroll=False)


@functools.partial(
    pl.kernel,
    out_type=jax.ShapeDtypeStruct((B * NRNG * 16,), jnp.int32),
    mesh=MESH,
    compiler_params=CP,
    scratch_types=[
        pltpu.VMEM((RSZ,), jnp.int32),
        pltpu.VMEM((RSZ,), jnp.int32),
        pltpu.VMEM((16,), jnp.int32),
    ],
)
def _k2a(cnt_i, tot_o, ba, bb, t16):
    wid = _wid()

    def batch(b, _):
        def rng(k, _):
            r = wid + k * 32
            pltpu.sync_copy(cnt_i.at[0, b, pl.ds(r * RSZ, RSZ)], ba)
            pltpu.sync_copy(cnt_i.at[1, b, pl.ds(r * RSZ, RSZ)], bb)

            def scan(j, acc):
                ab = ba[pl.ds(j * 16, 16)] + bb[pl.ds(j * 16, 16)]
                return acc + (ab > 0).astype(jnp.int32)

            acc = lax.fori_loop(0, RSZ // 16, scan, jnp.zeros((16,), jnp.int32),
                                unroll=False)
            t16[...] = jnp.full((16,), jnp.sum(acc), jnp.int32)
            pltpu.sync_copy(t16, tot_o.at[pl.ds((b * NRNG + r) * 16, 16)])
            return 0

        lax.fori_loop(0, 8, rng, 0, unroll=False)
        return 0

    lax.fori_loop(0, B, batch, 0, unroll=False)


@functools.partial(
    pl.kernel,
    out_type=(
        jax.ShapeDtypeStruct((B * BIGP,), jnp.int32),
        jax.ShapeDtypeStruct((B * SLOTP * 8,), jnp.int32),
        jax.ShapeDtypeStruct((B * SLOTP * 8,), jnp.int32),
    ),
    mesh=MESH,
    compiler_params=CP,
    scratch_types=[
        pltpu.VMEM((4096,), jnp.int32),
        pltpu.VMEM((256,), jnp.int32),
        pltpu.VMEM((RSZ,), jnp.int32),
        pltpu.VMEM((RSZ,), jnp.int32),
        pltpu.VMEM((RSZ,), jnp.int32),
        pltpu.VMEM((4096,), jnp.int32),
        pltpu.VMEM((4096,), jnp.int32),
        pltpu.VMEM((4096,), jnp.int32),
    ],
)
def _k2b(cnt_i, tot_i, sd_o, cp_o, vc_o, tbuf, pbuf, ba, bb, sdb, infb,
         cslab, vslab):
    wid = _wid()
    i16 = _i16()

    def init(i, _):
        infb[pl.ds(i * 16, 16)] = jnp.full((16,), INF, jnp.int32)
        return 0

    lax.fori_loop(0, 256, init, 0, unroll=False)

    def batch(b, _):
        pltpu.sync_copy(tot_i.at[pl.ds(b * NRNG * 16, 4096)], tbuf)

        def pfx(j, carry):
            tv = plsc.load_gather(tbuf, [(j * 16 + i16) * 16])
            cs = plsc.cumsum(tv)
            pbuf[pl.ds(j * 16, 16)] = carry + cs - tv
            return carry + jnp.max(cs)

        total = lax.fori_loop(0, 16, pfx, jnp.int32(0), unroll=False)

        def rng(k, _):
            r = wid + k * 32
            base = _extract(pbuf, r)

            @pl.when(base >= MAX_VOX)
            def _():
                def fi(j, _):
                    pltpu.sync_copy(
                        infb,
                        sd_o.at[pl.ds(b * BIGP + r * RSZ + j * 4096, 4096)])
                    return 0

                lax.fori_loop(0, RSZ // 4096, fi, 0, unroll=False)

            @pl.when(base < MAX_VOX)
            def _():
                pltpu.sync_copy(cnt_i.at[0, b, pl.ds(r * RSZ, RSZ)], ba)
                pltpu.sync_copy(cnt_i.at[1, b, pl.ds(r * RSZ, RSZ)], bb)

                def scan(j, carry):
                    ab = ba[pl.ds(j * 16, 16)] + bb[pl.ds(j * 16, 16)]
                    occ = (ab > 0).astype(jnp.int32)
                    cs = plsc.cumsum(occ)
                    slot = base + carry + cs - occ
                    sdv = jnp.where((occ > 0) & (slot < MAX_VOX), slot, INF)
                    sdb[pl.ds(j * 16, 16)] = sdv
                    return carry + jnp.max(cs)

                lax.fori_loop(0, RSZ // 16, scan, jnp.int32(0), unroll=False)
                pltpu.sync_copy(sdb, sd_o.at[pl.ds(b * BIGP + r * RSZ, RSZ)])
            return 0

        lax.fori_loop(0, 8, rng, 0, unroll=False)

        @pl.when(wid == 0)
        def _():
            pltpu.sync_copy(infb.at[pl.ds(0, 32)],
                            sd_o.at[pl.ds(b * BIGP + BIG, 32)])

        def zs(i, _):
            cslab[pl.ds(i * 16, 16)] = jnp.zeros((16,), jnp.int32)
            vslab[pl.ds(i * 16, 16)] = jnp.zeros((16,), jnp.int32)
            return 0

        lax.fori_loop(0, 256, zs, 0, unroll=False)
        slo = wid * SW
        shi = slo + SW

        @pl.when(slo < total)
        def _():
            def rng2(r, _):
                base = _extract(pbuf, r)
                tr = _extract(tbuf, r * 16)

                @pl.when((base < shi) & (base + tr > slo))
                def _():
                    pltpu.sync_copy(cnt_i.at[0, b, pl.ds(r * RSZ, RSZ)], ba)
                    pltpu.sync_copy(cnt_i.at[1, b, pl.ds(r * RSZ, RSZ)], bb)

                    def scan(j, carry):
                        ab = ba[pl.ds(j * 16, 16)] + bb[pl.ds(j * 16, 16)]
                        occ = (ab > 0).astype(jnp.int32)
                        cs = plsc.cumsum(occ)
                        slot = base + carry + cs - occ
                        mine = ((occ > 0) & (slot >= slo) & (slot < shi)
                                & (slot < MAX_VOX))
                        loc8 = (slot - slo) * 8
                        lv = r * RSZ + j * 16 + i16
                        plsc.store_scatter(cslab, [loc8],
                                           jnp.minimum(ab, MAX_PTS), mask=mine)
                        plsc.store_scatter(vslab, [loc8], lv >> 16, mask=mine)
                        plsc.store_scatter(vslab, [loc8 + 1], (lv >> 8) & 255,
                                           mask=mine)
                        plsc.store_scatter(vslab, [loc8 + 2], lv & 255,
                                           mask=mine)
                        return carry + jnp.max(cs)

                    lax.fori_loop(0, RSZ // 16, scan, jnp.int32(0),
                                  unroll=False)
                return 0

            lax.fori_loop(0, NRNG, rng2, 0, unroll=False)

        pltpu.sync_copy(cslab, cp_o.at[pl.ds((b * SLOTP + slo) * 8, 4096)])
        pltpu.sync_copy(vslab, vc_o.at[pl.ds((b * SLOTP + slo) * 8, 4096)])
        return 0

    lax.fori_loop(0, B, batch, 0, unroll=False)


@functools.partial(
    pl.kernel,
    out_type=(
        jax.ShapeDtypeStruct((B, 32, CHUNK), jnp.int32),
        jax.ShapeDtypeStruct((B * 32 * 16,), jnp.int32),
    ),
    mesh=MESH,
    compiler_params=CP,
    scratch_types=[
        pltpu.VMEM((49, 128), jnp.int32),
        pltpu.VMEM((49, 128), jnp.int32),
        pltpu.VMEM((CHUNK,), jnp.int32),
        pltpu.VMEM((16,), jnp.int32),
        pltpu.SemaphoreType.DMA,
    ],
)
def _k3a(lin_i, sd_i, flt_o, fc_o, l2, sd2, fbuf, t16, sem):
    wid = _wid()
    i16 = _i16()

    def batch(b, _):
        pltpu.sync_copy(lin_i.at[b, wid], l2)

        def grow(j, _):
            pltpu.async_copy(sd_i.at[pl.ds(b * BIGP, BIGP)].at[l2.at[j]],
                             sd2.at[j], sem).wait()
            return 0

        lax.fori_loop(0, 49, grow, 0, unroll=False)

        def row(j, off):
            for v in range(8):
                o = j * 128 + v * 16
                sdv = sd2[j, pl.ds(v * 16, 16)]
                lv = l2[j, pl.ds(v * 16, 16)]
                keep = (sdv < MAX_VOX) & (lv < BIG)
                pidx = wid * CHUNK + o + i16
                packed = lax.shift_left(pidx, 14) | jnp.where(keep, sdv, 0)
                plsc.store_compressed(fbuf.at[pl.ds(off, 16)], packed,
                                      mask=keep)
                off = off + jnp.sum(keep.astype(jnp.int32))
            return off

        off = lax.fori_loop(0, 49, row, jnp.int32(0), unroll=False)
        pltpu.sync_copy(fbuf, flt_o.at[b, wid])
        t16[...] = jnp.full((16,), off, jnp.int32)
        pltpu.sync_copy(t16, fc_o.at[pl.ds((b * 32 + wid) * 16, 16)])
        return 0

    lax.fori_loop(0, B, batch, 0, unroll=False)


@functools.partial(
    pl.kernel,
    out_type=(
        jax.ShapeDtypeStruct((B * SLOTP * 16,), jnp.float32),
        jax.ShapeDtypeStruct((B * SLOTP * 8,), jnp.float32),
    ),
    mesh=MESH,
    compiler_params=CP,
    scratch_types=[
        pltpu.VMEM((8192,), jnp.float32),
        pltpu.VMEM((512,), jnp.int32),
        pltpu.VMEM((512,), jnp.int32),
        pltpu.VMEM((512,), jnp.int32),
        pltpu.VMEM((1568,), jnp.int32),
        pltpu.VMEM((1568,), jnp.int32),
        pltpu.VMEM((16,), jnp.float32),
        pltpu.VMEM((4096,), jnp.int32),
        pltpu.VMEM((4096,), jnp.float32),
        pltpu.SemaphoreType.DMA,
    ],
)
def _k3c(flt_i, fc_i, cp_i, feat_i, vf_o, enc_o, slab, cloc, sbuf, fc, kidx,
         kfeat, fb, cbuf, ebuf, sem):
    wid = _wid()
    i16 = _i16()
    slo = wid * SW

    def batch(b, _):
        def zs(i, _):
            slab[pl.ds(i * 16, 16)] = jnp.zeros((16,), jnp.float32)
            return 0

        lax.fori_loop(0, 512, zs, 0, unroll=False)

        def zc(i, _):
            cloc[pl.ds(i * 16, 16)] = jnp.zeros((16,), jnp.int32)
            return 0

        lax.fori_loop(0, 32, zc, 0, unroll=False)

        pltpu.sync_copy(fc_i.at[pl.ds(b * 512, 512)], fc)

        def chunk(c, koff):
            nc = _extract(fc, c * 16)

            def tile(t, koff):
                pltpu.sync_copy(flt_i.at[b, c, pl.ds(t * 512, 512)], sbuf)

                def vr(j, koff):
                    e = sbuf[pl.ds(j * 16, 16)]
                    pos = t * 512 + j * 16 + i16
                    sdv = e & 0x3FFF
                    pidx = lax.shift_right_logical(e, 14)
                    mine = (pos < nc) & (sdv >= slo) & (sdv < slo + SW)
                    sl = jnp.where(mine, sdv - slo, 511)
                    dup, last = plsc.scan_count(sl, mask=mine)
                    c0 = plsc.load_gather(cloc, [sl], mask=mine)
                    rank = c0 + dup - 1
                    keep = mine & (rank < MAX_PTS)
                    plsc.store_scatter(cloc, [sl],
                                       jnp.minimum(c0 + dup, MAX_PTS),
                                       mask=mine & last)
                    nk = jnp.sum(keep.astype(jnp.int32))

                    @pl.when(nk > 0)
                    def _():
                        plsc.store_compressed(
                            kidx.at[pl.ds(koff, 16)],
                            sl * 16 + rank * 5, mask=keep)
                        plsc.store_compressed(
                            kfeat.at[pl.ds(koff, 16)],
                            (b * NPAD + pidx) * 5, mask=keep)
                    return koff + nk

                return lax.fori_loop(0, 32, vr, koff, unroll=False)

            nt = (nc + 511) // 512
            return lax.fori_loop(0, nt, tile, koff, unroll=False)

        koff = lax.fori_loop(0, 32, chunk, jnp.int32(0), unroll=False)

        def grp(g, _):
            ki = kidx[pl.ds(g * 16, 16)]
            kf = kfeat[pl.ds(g * 16, 16)]
            valid = g * 16 + i16 < koff
            kfs = jnp.where(valid, kf, 0)
            pltpu.async_copy(feat_i.at[kfs], fb, sem).wait()
            plsc.store_scatter(slab, [jnp.where(valid, ki, 8191)], fb[...],
                               mask=valid)
            for c5 in range(1, 5):
                pltpu.async_copy(feat_i.at[kfs + c5], fb, sem).wait()
                plsc.store_scatter(slab, [jnp.where(valid, ki + c5, 8191)],
                                   fb[...], mask=valid)
            return 0

        lax.fori_loop(0, (koff + 15) // 16, grp, 0, unroll=False)
        pltpu.sync_copy(slab.at[pl.ds(0, 4096)],
                        vf_o.at[pl.ds((b * SLOTP + slo) * 16, 4096)])
        pltpu.sync_copy(slab.at[pl.ds(4096, 4096)],
                        vf_o.at[pl.ds((b * SLOTP + slo) * 16 + 4096, 4096)])

        pltpu.sync_copy(cp_i.at[pl.ds((b * SLOTP + slo) * 8, 4096)], cbuf)

        def enc(j, _):
            sl16 = j * 16 + i16
            cnt = plsc.load_gather(cbuf, [sl16 * 8])
            den = jnp.maximum(cnt, 1).astype(jnp.float32)
            for c5 in range(5):
                s = (plsc.load_gather(slab, [sl16 * 16 + c5])
                     + plsc.load_gather(slab, [sl16 * 16 + 5 + c5])
                     + plsc.load_gather(slab, [sl16 * 16 + 10 + c5]))
                plsc.store_scatter(ebuf, [sl16 * 8 + c5], s / den)
            return 0

        lax.fori_loop(0, 32, enc, 0, unroll=False)
        pltpu.sync_copy(ebuf, enc_o.at[pl.ds((b * SLOTP + slo) * 8, 4096)])
        return 0

    lax.fori_loop(0, B, batch, 0, unroll=False)


def kernel(sparse_cube, sparse_cube_dop, batch_size):
    cat = jnp.concatenate([sparse_cube, sparse_cube_dop[:, :, 3:4]], axis=-1)
    pad = NPAD - N
    catp = jnp.pad(cat, ((0, 0), (0, pad), (0, 0)), constant_values=-10.0)
    xc = catp[:, :, 0]
    yc = catp[:, :, 1]
    zc = catp[:, :, 2]
    feat_flat = catp.reshape(B * NPAD * 5)

    lin, cnt2 = _k1(xc, yc, zc)
    tot = _k2a(cnt2)
    sd, cp, vc = _k2b(cnt2, tot)
    flt, fcn = _k3a(lin, sd)
    vf_pad, enc_pad = _k3c(flt, fcn, cp, feat_flat)

    enc_pad = enc_pad.reshape(B, SLOTP, 8)
    vf_pad = vf_pad.reshape(B, SLOTP, 16)
    cp = cp.reshape(B, SLOTP, 8)
    vc = vc.reshape(B, SLOTP, 8)
    encoded = enc_pad[:, :MAX_VOX, :5].reshape(B * MAX_VOX, 5)
    voxel_features = vf_pad[:, :MAX_VOX, :15].reshape(B * MAX_VOX, MAX_PTS, 5)
    counts = cp[:, :MAX_VOX, 0].reshape(B * MAX_VOX)
    bcol = jnp.minimum(jnp.arange(B, dtype=jnp.int32),
                       jnp.asarray(batch_size - 1, jnp.int32))
    bcol = jnp.repeat(bcol, MAX_VOX)[:, None]
    voxel_coords = jnp.concatenate(
        [bcol, vc[:, :MAX_VOX, :3].reshape(B * MAX_VOX, 3)], axis=1)
    pts_idx = jnp.repeat(jnp.arange(B), N).astype(cat.dtype)
    points = jnp.concatenate([pts_idx[:, None], cat.reshape(B * N, 5)], axis=-1)
    return encoded, voxel_features, voxel_coords, counts, points


# speedup vs baseline: 34.3113x; 1.1104x over previous
import functools

import jax
import jax.numpy as jnp
from jax import lax
from jax.experimental import pallas as pl
from jax.experimental.pallas import tpu as pltpu
from jax.experimental.pallas import tpu_sc as plsc

VOX = 0.4
XMIN, YMIN, ZMIN = 0.0, -51.2, -4.0
NX, NY, NZ = 256, 256, 20
MAX_VOX = 16000
MAX_PTS = 3
BIG = NX * NY * NZ
BIGP = BIG + 32
B, N = 4, 200000
NPAD = 200704
CHUNK = NPAD // 32
NRNG = 256
RSZ = BIG // NRNG
SLOTP = 16384
SW = 512
INF = 2**31 - 1

MESH = plsc.VectorSubcoreMesh(core_axis_name="c", subcore_axis_name="s")
CP = pltpu.CompilerParams(needs_layout_passes=False)


def _wid():
    return lax.axis_index("s") * 2 + lax.axis_index("c")


def _i16():
    return lax.iota(jnp.int32, 16)


def _extract(buf, i):
    v = plsc.load_gather(buf, [jnp.full((16,), i, jnp.int32)])
    return jnp.max(v)


@functools.partial(
    pl.kernel,
    out_type=(
        jax.ShapeDtypeStruct((B, 32, 49, 128), jnp.int32),
        jax.ShapeDtypeStruct((2, B, BIG), jnp.int32),
    ),
    mesh=MESH,
    compiler_params=CP,
    scratch_types=[
        pltpu.VMEM_SHARED((BIGP,), jnp.int32),
        pltpu.VMEM((CHUNK,), jnp.float32),
        pltpu.VMEM((CHUNK,), jnp.float32),
        pltpu.VMEM((CHUNK,), jnp.float32),
        pltpu.VMEM((49, 128), jnp.int32),
        pltpu.VMEM((128,), jnp.int32),
        pltpu.VMEM((16384,), jnp.int32),
        pltpu.SemaphoreType.DMA,
    ],
)
def _k1(xc, yc, zc, lin_o, cnt_o, spc, xb, yb, zb, lin2, ones, zbuf, sem):
    cid = lax.axis_index("c")
    sid = lax.axis_index("s")
    wid = _wid()
    i16 = _i16()

    def init(i, _):
        zbuf[pl.ds(i * 16, 16)] = jnp.zeros((16,), jnp.int32)
        return 0

    lax.fori_loop(0, 1024, init, 0, unroll=False)
    for j in range(8):
        ones[pl.ds(j * 16, 16)] = jnp.ones((16,), jnp.int32)

    def coord(vb, off, lo):
        v = (vb[pl.ds(off, 16)] - lo) / jnp.float32(VOX)
        ci = v.astype(jnp.int32)
        ci = ci - (ci.astype(jnp.float32) > v).astype(jnp.int32)
        return ci

    def batch(b, _):
        plsc.subcore_barrier()
        def z5(j, _):
            pltpu.sync_copy(zbuf, spc.at[pl.ds(sid * 81920 + j * 16384, 16384)])
            return 0

        lax.fori_loop(0, 5, z5, 0, unroll=False)

        @pl.when(sid == 0)
        def _():
            pltpu.sync_copy(zbuf.at[pl.ds(0, 32)], spc.at[pl.ds(BIG, 32)])

        plsc.subcore_barrier()
        base = wid * CHUNK
        pltpu.sync_copy(xc.at[b, pl.ds(base, CHUNK)], xb)
        pltpu.sync_copy(yc.at[b, pl.ds(base, CHUNK)], yb)
        pltpu.sync_copy(zc.at[b, pl.ds(base, CHUNK)], zb)

        def row(j, _):
            for v in range(8):
                o = j * 128 + v * 16
                cx = coord(xb, o, jnp.float32(XMIN))
                cy = coord(yb, o, jnp.float32(YMIN))
                cz = coord(zb, o, jnp.float32(ZMIN))
                inr = ((cx >= 0) & (cx < NX) & (cy >= 0) & (cy < NY)
                       & (cz >= 0) & (cz < NZ))
                lv = (cz * NY + cy) * NX + cx
                lv = jnp.where(inr, lv, BIG + i16)
                lin2[j, pl.ds(v * 16, 16)] = lv
            return 0

        lax.fori_loop(0, 49, row, 0, unroll=False)

        def sadd(j, _):
            pltpu.sync_copy(ones, spc.at[lin2.at[j]], add=True)
            return 0

        lax.fori_loop(0, 49, sadd, 0, unroll=False)
        pltpu.sync_copy(lin2, lin_o.at[b, wid])
        plsc.subcore_barrier()
        pltpu.sync_copy(spc.at[pl.ds(sid * 81920, 81920)],
                        cnt_o.at[cid, b, pl.ds(sid * 81920, 81920)])
        return 0

    lax.fori_loop(0, B, batch, 0, unroll=False)


@functools.partial(
    pl.kernel,
    out_type=jax.ShapeDtypeStruct((B * NRNG * 16,), jnp.int32),
    mesh=MESH,
    compiler_params=CP,
    scratch_types=[
        pltpu.VMEM((RSZ,), jnp.int32),
        pltpu.VMEM((RSZ,), jnp.int32),
        pltpu.VMEM((16,), jnp.int32),
    ],
)
def _k2a(cnt_i, tot_o, ba, bb, t16):
    wid = _wid()

    def batch(b, _):
        def rng(k, _):
            r = wid + k * 32
            pltpu.sync_copy(cnt_i.at[0, b, pl.ds(r * RSZ, RSZ)], ba)
            pltpu.sync_copy(cnt_i.at[1, b, pl.ds(r * RSZ, RSZ)], bb)

            def scan(j, acc):
                ab = ba[pl.ds(j * 16, 16)] + bb[pl.ds(j * 16, 16)]
                return acc + (ab > 0).astype(jnp.int32)

            acc = lax.fori_loop(0, RSZ // 16, scan, jnp.zeros((16,), jnp.int32),
                                unroll=False)
            t16[...] = jnp.full((16,), jnp.sum(acc), jnp.int32)
            pltpu.sync_copy(t16, tot_o.at[pl.ds((b * NRNG + r) * 16, 16)])
            return 0

        lax.fori_loop(0, 8, rng, 0, unroll=False)
        return 0

    lax.fori_loop(0, B, batch, 0, unroll=False)


@functools.partial(
    pl.kernel,
    out_type=(
        jax.ShapeDtypeStruct((B * BIGP,), jnp.int32),
        jax.ShapeDtypeStruct((B * SLOTP * 8,), jnp.int32),
        jax.ShapeDtypeStruct((B * SLOTP * 8,), jnp.int32),
    ),
    mesh=MESH,
    compiler_params=CP,
    scratch_types=[
        pltpu.VMEM((4096,), jnp.int32),
        pltpu.VMEM((256,), jnp.int32),
        pltpu.VMEM((RSZ,), jnp.int32),
        pltpu.VMEM((RSZ,), jnp.int32),
        pltpu.VMEM((RSZ,), jnp.int32),
        pltpu.VMEM((4096,), jnp.int32),
        pltpu.VMEM((4096,), jnp.int32),
        pltpu.VMEM((4096,), jnp.int32),
    ],
)
def _k2b(cnt_i, tot_i, sd_o, cp_o, vc_o, tbuf, pbuf, ba, bb, sdb, infb,
         cslab, vslab):
    wid = _wid()
    i16 = _i16()

    def init(i, _):
        infb[pl.ds(i * 16, 16)] = jnp.full((16,), INF, jnp.int32)
        return 0

    lax.fori_loop(0, 256, init, 0, unroll=False)

    def batch(b, _):
        pltpu.sync_copy(tot_i.at[pl.ds(b * NRNG * 16, 4096)], tbuf)

        def pfx(j, carry):
            tv = plsc.load_gather(tbuf, [(j * 16 + i16) * 16])
            cs = plsc.cumsum(tv)
            pbuf[pl.ds(j * 16, 16)] = carry + cs - tv
            return carry + jnp.max(cs)

        total = lax.fori_loop(0, 16, pfx, jnp.int32(0), unroll=False)

        def rng(k, _):
            r = wid + k * 32
            base = _extract(pbuf, r)

            @pl.when(base >= MAX_VOX)
            def _():
                def fi(j, _):
                    pltpu.sync_copy(
                        infb,
                        sd_o.at[pl.ds(b * BIGP + r * RSZ + j * 4096, 4096)])
                    return 0

                lax.fori_loop(0, RSZ // 4096, fi, 0, unroll=False)

            @pl.when(base < MAX_VOX)
            def _():
                pltpu.sync_copy(cnt_i.at[0, b, pl.ds(r * RSZ, RSZ)], ba)
                pltpu.sync_copy(cnt_i.at[1, b, pl.ds(r * RSZ, RSZ)], bb)

                def scan(j, carry):
                    ab = ba[pl.ds(j * 16, 16)] + bb[pl.ds(j * 16, 16)]
                    occ = (ab > 0).astype(jnp.int32)
                    cs = plsc.cumsum(occ)
                    slot = base + carry + cs - occ
                    sdv = jnp.where((occ > 0) & (slot < MAX_VOX), slot, INF)
                    sdb[pl.ds(j * 16, 16)] = sdv
                    return carry + jnp.max(cs)

                lax.fori_loop(0, RSZ // 16, scan, jnp.int32(0), unroll=False)
                pltpu.sync_copy(sdb, sd_o.at[pl.ds(b * BIGP + r * RSZ, RSZ)])
            return 0

        lax.fori_loop(0, 8, rng, 0, unroll=False)

        @pl.when(wid == 0)
        def _():
            pltpu.sync_copy(infb.at[pl.ds(0, 32)],
                            sd_o.at[pl.ds(b * BIGP + BIG, 32)])

        def zs(i, _):
            cslab[pl.ds(i * 16, 16)] = jnp.zeros((16,), jnp.int32)
            vslab[pl.ds(i * 16, 16)] = jnp.zeros((16,), jnp.int32)
            return 0

        lax.fori_loop(0, 256, zs, 0, unroll=False)
        slo = wid * SW
        shi = slo + SW

        @pl.when(slo < total)
        def _():
            def rng2(r, _):
                base = _extract(pbuf, r)
                tr = _extract(tbuf, r * 16)

                @pl.when((base < shi) & (base + tr > slo))
                def _():
                    pltpu.sync_copy(cnt_i.at[0, b, pl.ds(r * RSZ, RSZ)], ba)
                    pltpu.sync_copy(cnt_i.at[1, b, pl.ds(r * RSZ, RSZ)], bb)

                    def scan(j, carry):
                        ab = ba[pl.ds(j * 16, 16)] + bb[pl.ds(j * 16, 16)]
                        occ = (ab > 0).astype(jnp.int32)
                        cs = plsc.cumsum(occ)
                        slot = base + carry + cs - occ
                        mine = ((occ > 0) & (slot >= slo) & (slot < shi)
                                & (slot < MAX_VOX))
                        loc8 = (slot - slo) * 8
                        lv = r * RSZ + j * 16 + i16
                        plsc.store_scatter(cslab, [loc8],
                                           jnp.minimum(ab, MAX_PTS), mask=mine)
                        plsc.store_scatter(vslab, [loc8], lv >> 16, mask=mine)
                        plsc.store_scatter(vslab, [loc8 + 1], (lv >> 8) & 255,
                                           mask=mine)
                        plsc.store_scatter(vslab, [loc8 + 2], lv & 255,
                                           mask=mine)
                        return carry + jnp.max(cs)

                    lax.fori_loop(0, RSZ // 16, scan, jnp.int32(0),
                                  unroll=False)
                return 0

            lax.fori_loop(0, NRNG, rng2, 0, unroll=False)

        pltpu.sync_copy(cslab, cp_o.at[pl.ds((b * SLOTP + slo) * 8, 4096)])
        pltpu.sync_copy(vslab, vc_o.at[pl.ds((b * SLOTP + slo) * 8, 4096)])
        return 0

    lax.fori_loop(0, B, batch, 0, unroll=False)


@functools.partial(
    pl.kernel,
    out_type=(
        jax.ShapeDtypeStruct((B, 32, CHUNK), jnp.int32),
        jax.ShapeDtypeStruct((B * 32 * 16,), jnp.int32),
    ),
    mesh=MESH,
    compiler_params=CP,
    scratch_types=[
        pltpu.VMEM((49, 128), jnp.int32),
        pltpu.VMEM((49, 128), jnp.int32),
        pltpu.VMEM((CHUNK,), jnp.int32),
        pltpu.VMEM((16,), jnp.int32),
        pltpu.SemaphoreType.DMA,
    ],
)
def _k3a(lin_i, sd_i, flt_o, fc_o, l2, sd2, fbuf, t16, sem):
    wid = _wid()
    i16 = _i16()

    def batch(b, _):
        pltpu.sync_copy(lin_i.at[b, wid], l2)

        descs = []
        for j in range(49):
            descs.append(
                pltpu.async_copy(sd_i.at[pl.ds(b * BIGP, BIGP)].at[l2.at[j]],
                                 sd2.at[j], sem))
        for d in descs:
            d.wait()

        def row(j, off):
            for v in range(8):
                o = j * 128 + v * 16
                sdv = sd2[j, pl.ds(v * 16, 16)]
                lv = l2[j, pl.ds(v * 16, 16)]
                keep = (sdv < MAX_VOX) & (lv < BIG)
                pidx = wid * CHUNK + o + i16
                packed = lax.shift_left(pidx, 14) | jnp.where(keep, sdv, 0)
                plsc.store_compressed(fbuf.at[pl.ds(off, 16)], packed,
                                      mask=keep)
                off = off + jnp.sum(keep.astype(jnp.int32))
            return off

        off = lax.fori_loop(0, 49, row, jnp.int32(0), unroll=False)
        pltpu.sync_copy(fbuf, flt_o.at[b, wid])
        t16[...] = jnp.full((16,), off, jnp.int32)
        pltpu.sync_copy(t16, fc_o.at[pl.ds((b * 32 + wid) * 16, 16)])
        return 0

    lax.fori_loop(0, B, batch, 0, unroll=False)


@functools.partial(
    pl.kernel,
    out_type=(
        jax.ShapeDtypeStruct((B * SLOTP * 16,), jnp.float32),
        jax.ShapeDtypeStruct((B * SLOTP * 8,), jnp.float32),
    ),
    mesh=MESH,
    compiler_params=CP,
    scratch_types=[
        pltpu.VMEM((8192,), jnp.float32),
        pltpu.VMEM((512,), jnp.int32),
        pltpu.VMEM((512,), jnp.int32),
        pltpu.VMEM((512,), jnp.int32),
        pltpu.VMEM((1568,), jnp.int32),
        pltpu.VMEM((1568,), jnp.int32),
        pltpu.VMEM((80,), jnp.float32),
        pltpu.VMEM((4096,), jnp.int32),
        pltpu.VMEM((4096,), jnp.float32),
        pltpu.SMEM((1,), jnp.int32),
        pltpu.SemaphoreType.DMA,
    ],
)
def _k3c(flt_i, fc_i, cp_i, feat_i, vf_o, enc_o, slab, cloc, sbuf, fc, kidx,
         kfeat, fb, cbuf, ebuf, kref, sem):
    wid = _wid()
    i16 = _i16()
    slo = wid * SW

    def batch(b, _):
        def zs(i, _):
            slab[pl.ds(i * 16, 16)] = jnp.zeros((16,), jnp.float32)
            return 0

        lax.fori_loop(0, 512, zs, 0, unroll=False)

        def zc(i, _):
            cloc[pl.ds(i * 16, 16)] = jnp.zeros((16,), jnp.int32)
            return 0

        lax.fori_loop(0, 32, zc, 0, unroll=False)

        pltpu.sync_copy(fc_i.at[pl.ds(b * 512, 512)], fc)

        kref[0] = jnp.int32(0)

        def chunk(c, _):
            nc = _extract(fc, c * 16)

            def tile(t, _):
                pltpu.sync_copy(flt_i.at[b, c, pl.ds(t * 512, 512)], sbuf)

                def vr(j, _):
                    e = sbuf[pl.ds(j * 16, 16)]
                    pos = t * 512 + j * 16 + i16
                    sdv = e & 0x3FFF
                    mine = (pos < nc) & (sdv >= slo) & (sdv < slo + SW)
                    nm = jnp.sum(mine.astype(jnp.int32))

                    @pl.when(nm > 0)
                    def _():
                        pidx = lax.shift_right_logical(e, 14)
                        sl = jnp.where(mine, sdv - slo, 511)
                        dup, last = plsc.scan_count(sl, mask=mine)
                        c0 = plsc.load_gather(cloc, [sl], mask=mine)
                        rank = c0 + dup - 1
                        keep = mine & (rank < MAX_PTS)
                        plsc.store_scatter(cloc, [sl],
                                           jnp.minimum(c0 + dup, MAX_PTS),
                                           mask=mine & last)
                        nk = jnp.sum(keep.astype(jnp.int32))
                        ko = kref[0]
                        plsc.store_compressed(
                            kidx.at[pl.ds(ko, 16)],
                            sl * 16 + rank * 5, mask=keep)
                        plsc.store_compressed(
                            kfeat.at[pl.ds(ko, 16)],
                            (b * NPAD + pidx) * 5, mask=keep)
                        kref[0] = ko + nk
                    return 0

                lax.fori_loop(0, 32, vr, 0, unroll=False)
                return 0

            nt = (nc + 511) // 512
            lax.fori_loop(0, nt, tile, 0, unroll=False)
            return 0

        lax.fori_loop(0, 32, chunk, 0, unroll=False)
        koff = kref[0]

        def grp(g, _):
            ki = kidx[pl.ds(g * 16, 16)]
            kf = kfeat[pl.ds(g * 16, 16)]
            valid = g * 16 + i16 < koff
            kfs = jnp.where(valid, kf, 0)
            descs = []
            for c5 in range(5):
                descs.append(pltpu.async_copy(feat_i.at[kfs + c5],
                                              fb.at[pl.ds(c5 * 16, 16)], sem))
            for d in descs:
                d.wait()
            for c5 in range(5):
                plsc.store_scatter(slab, [jnp.where(valid, ki + c5, 8191)],
                                   fb[pl.ds(c5 * 16, 16)], mask=valid)
            return 0

        lax.fori_loop(0, (koff + 15) // 16, grp, 0, unroll=False)
        pltpu.sync_copy(slab.at[pl.ds(0, 4096)],
                        vf_o.at[pl.ds((b * SLOTP + slo) * 16, 4096)])
        pltpu.sync_copy(slab.at[pl.ds(4096, 4096)],
                        vf_o.at[pl.ds((b * SLOTP + slo) * 16 + 4096, 4096)])

        pltpu.sync_copy(cp_i.at[pl.ds((b * SLOTP + slo) * 8, 4096)], cbuf)

        def enc(j, _):
            sl16 = j * 16 + i16
            cnt = plsc.load_gather(cbuf, [sl16 * 8])
            den = jnp.maximum(cnt, 1).astype(jnp.float32)
            for c5 in range(5):
                s = (plsc.load_gather(slab, [sl16 * 16 + c5])
                     + plsc.load_gather(slab, [sl16 * 16 + 5 + c5])
                     + plsc.load_gather(slab, [sl16 * 16 + 10 + c5]))
                plsc.store_scatter(ebuf, [sl16 * 8 + c5], s / den)
            return 0

        lax.fori_loop(0, 32, enc, 0, unroll=False)
        pltpu.sync_copy(ebuf, enc_o.at[pl.ds((b * SLOTP + slo) * 8, 4096)])
        return 0

    lax.fori_loop(0, B, batch, 0, unroll=False)


def kernel(sparse_cube, sparse_cube_dop, batch_size):
    cat = jnp.concatenate([sparse_cube, sparse_cube_dop[:, :, 3:4]], axis=-1)
    pad = NPAD - N
    catp = jnp.pad(cat, ((0, 0), (0, pad), (0, 0)), constant_values=-10.0)
    xc = catp[:, :, 0]
    yc = catp[:, :, 1]
    zc = catp[:, :, 2]
    feat_flat = catp.reshape(B * NPAD * 5)

    lin, cnt2 = _k1(xc, yc, zc)
    tot = _k2a(cnt2)
    sd, cp, vc = _k2b(cnt2, tot)
    flt, fcn = _k3a(lin, sd)
    vf_pad, enc_pad = _k3c(flt, fcn, cp, feat_flat)

    enc_pad = enc_pad.reshape(B, SLOTP, 8)
    vf_pad = vf_pad.reshape(B, SLOTP, 16)
    cp = cp.reshape(B, SLOTP, 8)
    vc = vc.reshape(B, SLOTP, 8)
    encoded = enc_pad[:, :MAX_VOX, :5].reshape(B * MAX_VOX, 5)
    voxel_features = vf_pad[:, :MAX_VOX, :15].reshape(B * MAX_VOX, MAX_PTS, 5)
    counts = cp[:, :MAX_VOX, 0].reshape(B * MAX_VOX)
    bcol = jnp.minimum(jnp.arange(B, dtype=jnp.int32),
                       jnp.asarray(batch_size - 1, jnp.int32))
    bcol = jnp.repeat(bcol, MAX_VOX)[:, None]
    voxel_coords = jnp.concatenate(
        [bcol, vc[:, :MAX_VOX, :3].reshape(B * MAX_VOX, 3)], axis=1)
    pts_idx = jnp.repeat(jnp.arange(B), N).astype(cat.dtype)
    points = jnp.concatenate([pts_idx[:, None], cat.reshape(B * N, 5)], axis=-1)
    return encoded, voxel_features, voxel_coords, counts, points


# speedup vs baseline: 42.7060x; 1.2447x over previous
import functools

import jax
import jax.numpy as jnp
from jax import lax
from jax.experimental import pallas as pl
from jax.experimental.pallas import tpu as pltpu
from jax.experimental.pallas import tpu_sc as plsc

VOX = 0.4
XMIN, YMIN, ZMIN = 0.0, -51.2, -4.0
NX, NY, NZ = 256, 256, 20
MAX_VOX = 16000
MAX_PTS = 3
BIG = NX * NY * NZ
BIGP = BIG + 32
B, N = 4, 200000
NPAD = 200704
CHUNK = NPAD // 32
NRNG = 256
RSZ = BIG // NRNG
SLOTP = 16384
SW = 512
INF = 2**31 - 1

MESH = plsc.VectorSubcoreMesh(core_axis_name="c", subcore_axis_name="s")
CP = pltpu.CompilerParams(needs_layout_passes=False)


def _wid():
    return lax.axis_index("s") * 2 + lax.axis_index("c")


def _i16():
    return lax.iota(jnp.int32, 16)


def _extract(buf, i):
    v = plsc.load_gather(buf, [jnp.full((16,), i, jnp.int32)])
    return jnp.max(v)


@functools.partial(
    pl.kernel,
    out_type=(
        jax.ShapeDtypeStruct((B, 32, 49, 128), jnp.int32),
        jax.ShapeDtypeStruct((2, B, BIG), jnp.int32),
    ),
    mesh=MESH,
    compiler_params=CP,
    scratch_types=[
        pltpu.VMEM_SHARED((BIGP,), jnp.int32),
        pltpu.VMEM((CHUNK,), jnp.float32),
        pltpu.VMEM((CHUNK,), jnp.float32),
        pltpu.VMEM((CHUNK,), jnp.float32),
        pltpu.VMEM((49, 128), jnp.int32),
        pltpu.VMEM((128,), jnp.int32),
        pltpu.VMEM((16384,), jnp.int32),
        pltpu.SemaphoreType.DMA,
    ],
)
def _k1(xc, yc, zc, lin_o, cnt_o, spc, xb, yb, zb, lin2, ones, zbuf, sem):
    cid = lax.axis_index("c")
    sid = lax.axis_index("s")
    wid = _wid()
    i16 = _i16()

    def init(i, _):
        zbuf[pl.ds(i * 16, 16)] = jnp.zeros((16,), jnp.int32)
        return 0

    lax.fori_loop(0, 1024, init, 0, unroll=False)
    for j in range(8):
        ones[pl.ds(j * 16, 16)] = jnp.ones((16,), jnp.int32)

    def coord(vb, off, lo):
        v = (vb[pl.ds(off, 16)] - lo) / jnp.float32(VOX)
        ci = v.astype(jnp.int32)
        ci = ci - (ci.astype(jnp.float32) > v).astype(jnp.int32)
        return ci

    def batch(b, _):
        plsc.subcore_barrier()
        def z5(j, _):
            pltpu.sync_copy(zbuf, spc.at[pl.ds(sid * 81920 + j * 16384, 16384)])
            return 0

        lax.fori_loop(0, 5, z5, 0, unroll=False)

        @pl.when(sid == 0)
        def _():
            pltpu.sync_copy(zbuf.at[pl.ds(0, 32)], spc.at[pl.ds(BIG, 32)])

        plsc.subcore_barrier()
        base = wid * CHUNK
        pltpu.sync_copy(xc.at[b, pl.ds(base, CHUNK)], xb)
        pltpu.sync_copy(yc.at[b, pl.ds(base, CHUNK)], yb)
        pltpu.sync_copy(zc.at[b, pl.ds(base, CHUNK)], zb)

        def row(j, _):
            for v in range(8):
                o = j * 128 + v * 16
                cx = coord(xb, o, jnp.float32(XMIN))
                cy = coord(yb, o, jnp.float32(YMIN))
                cz = coord(zb, o, jnp.float32(ZMIN))
                inr = ((cx >= 0) & (cx < NX) & (cy >= 0) & (cy < NY)
                       & (cz >= 0) & (cz < NZ))
                lv = (cz * NY + cy) * NX + cx
                lv = jnp.where(inr, lv, BIG + i16)
                lin2[j, pl.ds(v * 16, 16)] = lv
            return 0

        lax.fori_loop(0, 49, row, 0, unroll=False)

        def sadd(j, _):
            pltpu.sync_copy(ones, spc.at[lin2.at[j]], add=True)
            return 0

        lax.fori_loop(0, 49, sadd, 0, unroll=False)
        pltpu.sync_copy(lin2, lin_o.at[b, wid])
        plsc.subcore_barrier()
        pltpu.sync_copy(spc.at[pl.ds(sid * 81920, 81920)],
                        cnt_o.at[cid, b, pl.ds(sid * 81920, 81920)])
        return 0

    lax.fori_loop(0, B, batch, 0, unroll=False)


@functools.partial(
    pl.kernel,
    out_type=jax.ShapeDtypeStruct((B * NRNG * 16,), jnp.int32),
    mesh=MESH,
    compiler_params=CP,
    scratch_types=[
        pltpu.VMEM((RSZ,), jnp.int32),
        pltpu.VMEM((RSZ,), jnp.int32),
        pltpu.VMEM((16,), jnp.int32),
    ],
)
def _k2a(cnt_i, tot_o, ba, bb, t16):
    wid = _wid()

    def batch(b, _):
        def rng(k, _):
            r = wid + k * 32
            pltpu.sync_copy(cnt_i.at[0, b, pl.ds(r * RSZ, RSZ)], ba)
            pltpu.sync_copy(cnt_i.at[1, b, pl.ds(r * RSZ, RSZ)], bb)

            def scan(j, acc):
                ab = ba[pl.ds(j * 16, 16)] + bb[pl.ds(j * 16, 16)]
                return acc + (ab > 0).astype(jnp.int32)

            acc = lax.fori_loop(0, RSZ // 16, scan, jnp.zeros((16,), jnp.int32),
                                unroll=False)
            t16[...] = jnp.full((16,), jnp.sum(acc), jnp.int32)
            pltpu.sync_copy(t16, tot_o.at[pl.ds((b * NRNG + r) * 16, 16)])
            return 0

        lax.fori_loop(0, 8, rng, 0, unroll=False)
        return 0

    lax.fori_loop(0, B, batch, 0, unroll=False)


@functools.partial(
    pl.kernel,
    out_type=(
        jax.ShapeDtypeStruct((B * BIGP,), jnp.int32),
        jax.ShapeDtypeStruct((B * SLOTP * 8,), jnp.int32),
        jax.ShapeDtypeStruct((B * SLOTP * 8,), jnp.int32),
    ),
    mesh=MESH,
    compiler_params=CP,
    scratch_types=[
        pltpu.VMEM((4096,), jnp.int32),
        pltpu.VMEM((256,), jnp.int32),
        pltpu.VMEM((RSZ,), jnp.int32),
        pltpu.VMEM((RSZ,), jnp.int32),
        pltpu.VMEM((RSZ,), jnp.int32),
        pltpu.VMEM((4096,), jnp.int32),
        pltpu.VMEM((4096,), jnp.int32),
        pltpu.VMEM((4096,), jnp.int32),
    ],
)
def _k2b(cnt_i, tot_i, sd_o, cp_o, vc_o, tbuf, pbuf, ba, bb, sdb, infb,
         cslab, vslab):
    wid = _wid()
    i16 = _i16()

    def init(i, _):
        infb[pl.ds(i * 16, 16)] = jnp.full((16,), INF, jnp.int32)
        return 0

    lax.fori_loop(0, 256, init, 0, unroll=False)

    def batch(b, _):
        pltpu.sync_copy(tot_i.at[pl.ds(b * NRNG * 16, 4096)], tbuf)

        def pfx(j, carry):
            tv = plsc.load_gather(tbuf, [(j * 16 + i16) * 16])
            cs = plsc.cumsum(tv)
            pbuf[pl.ds(j * 16, 16)] = carry + cs - tv
            return carry + jnp.max(cs)

        total = lax.fori_loop(0, 16, pfx, jnp.int32(0), unroll=False)

        def rng(k, _):
            r = wid + k * 32
            base = _extract(pbuf, r)

            @pl.when(base >= MAX_VOX)
            def _():
                def fi(j, _):
                    pltpu.sync_copy(
                        infb,
                        sd_o.at[pl.ds(b * BIGP + r * RSZ + j * 4096, 4096)])
                    return 0

                lax.fori_loop(0, RSZ // 4096, fi, 0, unroll=False)

            @pl.when(base < MAX_VOX)
            def _():
                pltpu.sync_copy(cnt_i.at[0, b, pl.ds(r * RSZ, RSZ)], ba)
                pltpu.sync_copy(cnt_i.at[1, b, pl.ds(r * RSZ, RSZ)], bb)

                def scan(j, carry):
                    ab = ba[pl.ds(j * 16, 16)] + bb[pl.ds(j * 16, 16)]
                    occ = (ab > 0).astype(jnp.int32)
                    cs = plsc.cumsum(occ)
                    slot = base + carry + cs - occ
                    sdv = jnp.where((occ > 0) & (slot < MAX_VOX), slot, INF)
                    sdb[pl.ds(j * 16, 16)] = sdv
                    return carry + jnp.max(cs)

                lax.fori_loop(0, RSZ // 16, scan, jnp.int32(0), unroll=False)
                pltpu.sync_copy(sdb, sd_o.at[pl.ds(b * BIGP + r * RSZ, RSZ)])
            return 0

        lax.fori_loop(0, 8, rng, 0, unroll=False)

        @pl.when(wid == 0)
        def _():
            pltpu.sync_copy(infb.at[pl.ds(0, 32)],
                            sd_o.at[pl.ds(b * BIGP + BIG, 32)])

        def zs(i, _):
            cslab[pl.ds(i * 16, 16)] = jnp.zeros((16,), jnp.int32)
            vslab[pl.ds(i * 16, 16)] = jnp.zeros((16,), jnp.int32)
            return 0

        lax.fori_loop(0, 256, zs, 0, unroll=False)
        slo = wid * SW
        shi = slo + SW

        @pl.when(slo < total)
        def _():
            def rng2(r, _):
                base = _extract(pbuf, r)
                tr = _extract(tbuf, r * 16)

                @pl.when((base < shi) & (base + tr > slo))
                def _():
                    pltpu.sync_copy(cnt_i.at[0, b, pl.ds(r * RSZ, RSZ)], ba)
                    pltpu.sync_copy(cnt_i.at[1, b, pl.ds(r * RSZ, RSZ)], bb)

                    def scan(j, carry):
                        ab = ba[pl.ds(j * 16, 16)] + bb[pl.ds(j * 16, 16)]
                        occ = (ab > 0).astype(jnp.int32)
                        cs = plsc.cumsum(occ)
                        slot = base + carry + cs - occ
                        mine = ((occ > 0) & (slot >= slo) & (slot < shi)
                                & (slot < MAX_VOX))
                        loc8 = (slot - slo) * 8
                        lv = r * RSZ + j * 16 + i16
                        plsc.store_scatter(cslab, [loc8],
                                           jnp.minimum(ab, MAX_PTS), mask=mine)
                        plsc.store_scatter(vslab, [loc8], lv >> 16, mask=mine)
                        plsc.store_scatter(vslab, [loc8 + 1], (lv >> 8) & 255,
                                           mask=mine)
                        plsc.store_scatter(vslab, [loc8 + 2], lv & 255,
                                           mask=mine)
                        return carry + jnp.max(cs)

                    lax.fori_loop(0, RSZ // 16, scan, jnp.int32(0),
                                  unroll=False)
                return 0

            lax.fori_loop(0, NRNG, rng2, 0, unroll=False)

        pltpu.sync_copy(cslab, cp_o.at[pl.ds((b * SLOTP + slo) * 8, 4096)])
        pltpu.sync_copy(vslab, vc_o.at[pl.ds((b * SLOTP + slo) * 8, 4096)])
        return 0

    lax.fori_loop(0, B, batch, 0, unroll=False)


@functools.partial(
    pl.kernel,
    out_type=(
        jax.ShapeDtypeStruct((B * 32 * CHUNK,), jnp.int32),
        jax.ShapeDtypeStruct((B * 32 * 48,), jnp.int32),
    ),
    mesh=MESH,
    compiler_params=CP,
    scratch_types=[
        pltpu.VMEM((49, 128), jnp.int32),
        pltpu.VMEM((49, 128), jnp.int32),
        pltpu.VMEM((CHUNK,), jnp.int32),
        pltpu.VMEM((CHUNK,), jnp.int32),
        pltpu.VMEM((32,), jnp.int32),
        pltpu.VMEM((48,), jnp.int32),
        pltpu.SemaphoreType.DMA,
    ],
)
def _k3a(lin_i, sd_i, flt_o, of_o, l2, sd2, fbuf, fbuf2, run, offs, sem):
    wid = _wid()
    i16 = _i16()

    def batch(b, _):
        pltpu.sync_copy(lin_i.at[b, wid], l2)

        descs = []
        for j in range(49):
            descs.append(
                pltpu.async_copy(sd_i.at[pl.ds(b * BIGP, BIGP)].at[l2.at[j]],
                                 sd2.at[j], sem))
        for d in descs:
            d.wait()

        def row(j, off):
            for v in range(8):
                o = j * 128 + v * 16
                sdv = sd2[j, pl.ds(v * 16, 16)]
                lv = l2[j, pl.ds(v * 16, 16)]
                keep = (sdv < MAX_VOX) & (lv < BIG)
                pidx = wid * CHUNK + o + i16
                packed = lax.shift_left(pidx, 14) | jnp.where(keep, sdv, 0)
                plsc.store_compressed(fbuf.at[pl.ds(off, 16)], packed,
                                      mask=keep)
                off = off + jnp.sum(keep.astype(jnp.int32))
            return off

        off = lax.fori_loop(0, 49, row, jnp.int32(0), unroll=False)

        for j in range(2):
            run[pl.ds(j * 16, 16)] = jnp.zeros((16,), jnp.int32)

        def cntv(k, _):
            e = fbuf[pl.ds(k * 16, 16)]
            valid = k * 16 + i16 < off
            ow = jnp.where(valid, (e & 0x3FFF) >> 9, 31)
            dup, last = plsc.scan_count(ow, mask=valid)
            c0 = plsc.load_gather(run, [ow], mask=valid)
            plsc.store_scatter(run, [ow], c0 + dup, mask=valid & last)
            return 0

        nv = (off + 15) // 16
        lax.fori_loop(0, nv, cntv, 0, unroll=False)
        cl = run[pl.ds(0, 16)]
        ch = run[pl.ds(16, 16)]
        tl = jnp.sum(cl)
        csl = plsc.cumsum(cl)
        csh = plsc.cumsum(ch) + tl
        exl = csl - cl
        exh = csh - ch
        run[pl.ds(0, 16)] = exl
        run[pl.ds(16, 16)] = exh
        offs[pl.ds(0, 16)] = exl
        offs[pl.ds(16, 16)] = exh
        offs[pl.ds(32, 16)] = jnp.full((16,), off, jnp.int32)

        def mov(k, _):
            e = fbuf[pl.ds(k * 16, 16)]
            valid = k * 16 + i16 < off
            ow = jnp.where(valid, (e & 0x3FFF) >> 9, 31)
            dup, last = plsc.scan_count(ow, mask=valid)
            p0 = plsc.load_gather(run, [ow], mask=valid)
            pos = jnp.where(valid, p0 + dup - 1, CHUNK - 1)
            plsc.store_scatter(fbuf2, [pos], e, mask=valid)
            plsc.store_scatter(run, [ow], p0 + dup, mask=valid & last)
            return 0

        lax.fori_loop(0, nv, mov, 0, unroll=False)
        pltpu.sync_copy(fbuf2, flt_o.at[pl.ds((b * 32 + wid) * CHUNK, CHUNK)])
        pltpu.sync_copy(offs, of_o.at[pl.ds((b * 32 + wid) * 48, 48)])
        return 0

    lax.fori_loop(0, B, batch, 0, unroll=False)


@functools.partial(
    pl.kernel,
    out_type=(
        jax.ShapeDtypeStruct((B * SLOTP * 16,), jnp.float32),
        jax.ShapeDtypeStruct((B * SLOTP * 8,), jnp.float32),
    ),
    mesh=MESH,
    compiler_params=CP,
    scratch_types=[
        pltpu.VMEM((8192,), jnp.float32),
        pltpu.VMEM((512,), jnp.int32),
        pltpu.VMEM((1536,), jnp.int32),
        pltpu.VMEM((1, 128), jnp.int32),
        pltpu.VMEM((1, 128), jnp.int32),
        pltpu.VMEM((1568,), jnp.int32),
        pltpu.VMEM((1568,), jnp.int32),
        pltpu.VMEM((80,), jnp.float32),
        pltpu.VMEM((4096,), jnp.int32),
        pltpu.VMEM((4096,), jnp.float32),
        pltpu.SMEM((1,), jnp.int32),
        pltpu.SemaphoreType.DMA,
    ],
)
def _k3c(flt_i, of_i, cp_i, feat_i, vf_o, enc_o, slab, cloc, ofs, gI2, sgb,
         kidx, kfeat, fb, cbuf, ebuf, kref, sem):
    wid = _wid()
    i16 = _i16()
    slo = wid * SW

    def batch(b, _):
        def zs(i, _):
            slab[pl.ds(i * 16, 16)] = jnp.zeros((16,), jnp.float32)
            return 0

        lax.fori_loop(0, 512, zs, 0, unroll=False)

        def zc(i, _):
            cloc[pl.ds(i * 16, 16)] = jnp.zeros((16,), jnp.int32)
            return 0

        lax.fori_loop(0, 32, zc, 0, unroll=False)

        pltpu.sync_copy(of_i.at[pl.ds(b * 32 * 48, 1536)], ofs)

        kref[0] = jnp.int32(0)

        def chunk(c, _):
            o0 = _extract(ofs, c * 48 + wid)
            o1 = _extract(ofs, c * 48 + wid + 1)
            ncw = o1 - o0

            def tile(t, _):
                for v in range(8):
                    gI2[0, pl.ds(v * 16, 16)] = jnp.minimum(
                        o0 + t * 128 + v * 16 + i16, CHUNK - 1)
                pltpu.async_copy(
                    flt_i.at[pl.ds((b * 32 + c) * CHUNK, CHUNK)].at[gI2.at[0]],
                    sgb.at[0], sem).wait()
                nj = jnp.minimum((ncw - t * 128 + 15) // 16, 8)

                def vr(j, _):
                    e = sgb[0, pl.ds(j * 16, 16)]
                    valid = t * 128 + j * 16 + i16 < ncw
                    sdv = e & 0x3FFF
                    pidx = lax.shift_right_logical(e, 14)
                    sl = jnp.where(valid, sdv - slo, 511)
                    dup, last = plsc.scan_count(sl, mask=valid)
                    c0 = plsc.load_gather(cloc, [sl], mask=valid)
                    rank = c0 + dup - 1
                    keep = valid & (rank < MAX_PTS)
                    plsc.store_scatter(cloc, [sl],
                                       jnp.minimum(c0 + dup, MAX_PTS),
                                       mask=valid & last)
                    nk = jnp.sum(keep.astype(jnp.int32))
                    ko = kref[0]
                    plsc.store_compressed(kidx.at[pl.ds(ko, 16)],
                                          sl * 16 + rank * 5, mask=keep)
                    plsc.store_compressed(kfeat.at[pl.ds(ko, 16)],
                                          (b * NPAD + pidx) * 5, mask=keep)
                    kref[0] = ko + nk
                    return 0

                lax.fori_loop(0, nj, vr, 0, unroll=False)
                return 0

            nt = (ncw + 127) // 128
            lax.fori_loop(0, nt, tile, 0, unroll=False)
            return 0

        lax.fori_loop(0, 32, chunk, 0, unroll=False)
        koff = kref[0]

        def grp(g, _):
            ki = kidx[pl.ds(g * 16, 16)]
            kf = kfeat[pl.ds(g * 16, 16)]
            valid = g * 16 + i16 < koff
            kfs = jnp.where(valid, kf, 0)
            descs = []
            for c5 in range(5):
                descs.append(pltpu.async_copy(feat_i.at[kfs + c5],
                                              fb.at[pl.ds(c5 * 16, 16)], sem))
            for d in descs:
                d.wait()
            for c5 in range(5):
                plsc.store_scatter(slab, [jnp.where(valid, ki + c5, 8191)],
                                   fb[pl.ds(c5 * 16, 16)], mask=valid)
            return 0

        lax.fori_loop(0, (koff + 15) // 16, grp, 0, unroll=False)
        pltpu.sync_copy(slab.at[pl.ds(0, 4096)],
                        vf_o.at[pl.ds((b * SLOTP + slo) * 16, 4096)])
        pltpu.sync_copy(slab.at[pl.ds(4096, 4096)],
                        vf_o.at[pl.ds((b * SLOTP + slo) * 16 + 4096, 4096)])

        pltpu.sync_copy(cp_i.at[pl.ds((b * SLOTP + slo) * 8, 4096)], cbuf)

        def enc(j, _):
            sl16 = j * 16 + i16
            cnt = plsc.load_gather(cbuf, [sl16 * 8])
            den = jnp.maximum(cnt, 1).astype(jnp.float32)
            for c5 in range(5):
                s = (plsc.load_gather(slab, [sl16 * 16 + c5])
                     + plsc.load_gather(slab, [sl16 * 16 + 5 + c5])
                     + plsc.load_gather(slab, [sl16 * 16 + 10 + c5]))
                plsc.store_scatter(ebuf, [sl16 * 8 + c5], s / den)
            return 0

        lax.fori_loop(0, 32, enc, 0, unroll=False)
        pltpu.sync_copy(ebuf, enc_o.at[pl.ds((b * SLOTP + slo) * 8, 4096)])
        return 0

    lax.fori_loop(0, B, batch, 0, unroll=False)


def kernel(sparse_cube, sparse_cube_dop, batch_size):
    cat = jnp.concatenate([sparse_cube, sparse_cube_dop[:, :, 3:4]], axis=-1)
    pad = NPAD - N
    catp = jnp.pad(cat, ((0, 0), (0, pad), (0, 0)), constant_values=-10.0)
    xc = catp[:, :, 0]
    yc = catp[:, :, 1]
    zc = catp[:, :, 2]
    feat_flat = catp.reshape(B * NPAD * 5)

    lin, cnt2 = _k1(xc, yc, zc)
    tot = _k2a(cnt2)
    sd, cp, vc = _k2b(cnt2, tot)
    flt, ofc = _k3a(lin, sd)
    vf_pad, enc_pad = _k3c(flt, ofc, cp, feat_flat)

    enc_pad = enc_pad.reshape(B, SLOTP, 8)
    vf_pad = vf_pad.reshape(B, SLOTP, 16)
    cp = cp.reshape(B, SLOTP, 8)
    vc = vc.reshape(B, SLOTP, 8)
    encoded = enc_pad[:, :MAX_VOX, :5].reshape(B * MAX_VOX, 5)
    voxel_features = vf_pad[:, :MAX_VOX, :15].reshape(B * MAX_VOX, MAX_PTS, 5)
    counts = cp[:, :MAX_VOX, 0].reshape(B * MAX_VOX)
    bcol = jnp.minimum(jnp.arange(B, dtype=jnp.int32),
                       jnp.asarray(batch_size - 1, jnp.int32))
    bcol = jnp.repeat(bcol, MAX_VOX)[:, None]
    voxel_coords = jnp.concatenate(
        [bcol, vc[:, :MAX_VOX, :3].reshape(B * MAX_VOX, 3)], axis=1)
    pts_idx = jnp.repeat(jnp.arange(B), N).astype(cat.dtype)
    points = jnp.concatenate([pts_idx[:, None], cat.reshape(B * N, 5)], axis=-1)
    return encoded, voxel_features, voxel_coords, counts, points
